# Initial kernel scaffold; baseline (speedup 1.0000x reference)
import functools

import jax
import jax.numpy as jnp
from jax import lax
from jax.experimental import pallas as pl
from jax.experimental.pallas import tpu as pltpu
from jax.experimental.pallas import tpu_sc as plsc

BATCH = 16384
VOCAB = 32
D = 32
NCOL = 8
NW = 32
NQ = NW // NCOL
ROWS_W = BATCH // NQ
SUB = 128
NSUB = ROWS_W // SUB
IDX_ROWS = BATCH // SUB
IDX_ROWS_W = ROWS_W // SUB


def _body(c0, c1, c2, c3, c4, c5, c6, c7,
          w0, w1, w2, w3, w4, w5, w6, w7,
          out, idx_v, rows_v, sem):
  cs = (c0, c1, c2, c3, c4, c5, c6, c7)
  ws = (w0, w1, w2, w3, w4, w5, w6, w7)
  cid = lax.axis_index("c")
  sid = lax.axis_index("s")
  wid = sid * 2 + cid
  col = wid % NCOL
  q = wid // NCOL

  for j in range(NCOL):
    @pl.when(col == j)
    def _(j=j):
      pltpu.sync_copy(cs[j].at[pl.ds(q * IDX_ROWS_W, IDX_ROWS_W)], idx_v)
      row0 = q * ROWS_W

      def step(t, carry):
        pltpu.async_copy(ws[j].at[idx_v.at[t]], rows_v, sem).wait()
        pltpu.sync_copy(
            rows_v,
            out.at[pl.ds(row0 + t * SUB, SUB), pl.ds(j * D, D)])
        return carry

      lax.fori_loop(0, NSUB, step, 0)


@jax.jit
def _run(c0, c1, c2, c3, c4, c5, c6, c7,
         W_c0, W_c1, W_c2, W_c3, W_c4, W_c5, W_c6, W_c7):
  mesh = plsc.VectorSubcoreMesh(core_axis_name="c", subcore_axis_name="s")
  f = pl.kernel(
      _body,
      out_type=jax.ShapeDtypeStruct((BATCH, NCOL * D), jnp.float32),
      mesh=mesh,
      scratch_types=[
          pltpu.VMEM((IDX_ROWS_W, SUB), jnp.int32),
          pltpu.VMEM((SUB, D), jnp.float32),
          pltpu.SemaphoreType.DMA,
      ],
  )
  return f(c0, c1, c2, c3, c4, c5, c6, c7,
           W_c0, W_c1, W_c2, W_c3, W_c4, W_c5, W_c6, W_c7)


def kernel(c0, c1, c2, c3, c4, c5, c6, c7,
           W_c0, W_c1, W_c2, W_c3, W_c4, W_c5, W_c6, W_c7):
  cs = [c.reshape(IDX_ROWS, SUB) for c in (c0, c1, c2, c3, c4, c5, c6, c7)]
  return _run(*cs, W_c0, W_c1, W_c2, W_c3, W_c4, W_c5, W_c6, W_c7)


# speedup vs baseline: 6.8894x; 6.8894x over previous
import functools

import jax
import jax.numpy as jnp
from jax import lax
from jax.experimental import pallas as pl
from jax.experimental.pallas import tpu as pltpu
from jax.experimental.pallas import tpu_sc as plsc

BATCH = 16384
VOCAB = 32
D = 32
NCOL = 8
OUT_W = NCOL * D
NW = 32
ROWS_W = BATCH // NW
CH = 128
NCH = ROWS_W // CH
CHW = CH * OUT_W
TAB_W = NCOL * VOCAB * D


def _body(c0, c1, c2, c3, c4, c5, c6, c7, wtab, out,
          tab_v, i0, i1, i2, i3, i4, i5, i6, i7,
          buf_a, buf_b, sem_in, sem_a, sem_b):
  cs = (c0, c1, c2, c3, c4, c5, c6, c7)
  ivs = (i0, i1, i2, i3, i4, i5, i6, i7)
  wid = lax.axis_index("s") * 2 + lax.axis_index("c")
  b0 = wid * ROWS_W

  copies = [pltpu.async_copy(wtab, tab_v, sem_in)]
  for j in range(NCOL):
    copies.append(
        pltpu.async_copy(cs[j].at[pl.ds(b0, ROWS_W)], ivs[j], sem_in))
  for cp in copies:
    cp.wait()

  def fill(k, buf):
    def group(g, carry):
      cvecs = [ivs[j][pl.ds(pl.multiple_of((k * CH // 16 + g) * 16, 16), 16)]
               for j in range(NCOL)]
      r0 = pl.multiple_of(g * 16 * OUT_W, 16)
      for l in range(16):
        for j in range(NCOL):
          c = cvecs[j][l]
          base = pl.multiple_of(c * D + j * (VOCAB * D), D)
          dst = pl.multiple_of(r0 + l * OUT_W + j * D, D)
          for h in (0, 16):
            buf[pl.ds(dst + h, 16)] = tab_v[pl.ds(base + h, 16)]
      return carry

    lax.fori_loop(0, CH // 16, group, 0)

  def flush(k, buf, sem):
    return pltpu.async_copy(
        buf, out.at[pl.ds((b0 + k * CH) * OUT_W, CHW)], sem)

  fill(0, buf_a)
  d0 = flush(0, buf_a, sem_a)
  fill(1, buf_b)
  d1 = flush(1, buf_b, sem_b)
  d0.wait()
  fill(2, buf_a)
  d2 = flush(2, buf_a, sem_a)
  d1.wait()
  fill(3, buf_b)
  d3 = flush(3, buf_b, sem_b)
  d2.wait()
  d3.wait()


@jax.jit
def _run(c0, c1, c2, c3, c4, c5, c6, c7, wtab):
  mesh = plsc.VectorSubcoreMesh(core_axis_name="c", subcore_axis_name="s")
  f = pl.kernel(
      _body,
      out_type=jax.ShapeDtypeStruct((BATCH * OUT_W,), jnp.float32),
      mesh=mesh,
      compiler_params=pltpu.CompilerParams(needs_layout_passes=False),
      scratch_types=[
          pltpu.VMEM((TAB_W,), jnp.float32),
      ] + [pltpu.VMEM((ROWS_W,), jnp.int32)] * NCOL + [
          pltpu.VMEM((CHW,), jnp.float32),
          pltpu.VMEM((CHW,), jnp.float32),
          pltpu.SemaphoreType.DMA,
          pltpu.SemaphoreType.DMA,
          pltpu.SemaphoreType.DMA,
      ],
  )
  flat = f(c0, c1, c2, c3, c4, c5, c6, c7, wtab)
  return flat.reshape(BATCH, OUT_W)


def kernel(c0, c1, c2, c3, c4, c5, c6, c7,
           W_c0, W_c1, W_c2, W_c3, W_c4, W_c5, W_c6, W_c7):
  wtab = jnp.concatenate(
      [W_c0, W_c1, W_c2, W_c3, W_c4, W_c5, W_c6, W_c7], axis=0).reshape(TAB_W)
  return _run(c0, c1, c2, c3, c4, c5, c6, c7, wtab)
